# Initial kernel scaffold; baseline (speedup 1.0000x reference)
import jax
import jax.numpy as jnp
from jax.experimental import pallas as pl


def kernel(obj_l0, obj_l1, obj_l2, obj_l3, obj_l4, del_l0, del_l1, del_l2, del_l3, del_l4, anchors, image_sizes):
    raise NotImplementedError("write your pallas kernel here")


# speedup vs baseline: 8.1088x; 8.1088x over previous
import math

import jax
import jax.numpy as jnp
import numpy as np
from jax import lax
from jax.experimental import pallas as pl
from jax.experimental.pallas import tpu as pltpu

_N_IMGS = 2
_A = 3
_LEVEL_HW = [(128, 128), (64, 64), (32, 32), (16, 16), (8, 8)]
_PRE_NMS_TOP_N = 1000
_POST_NMS_TOP_N = 1000
_NMS_THRESH = 0.7
_SCORE_THRESH = 0.0
_MIN_SIZE = 0.001
_BBOX_XFORM_CLIP = math.log(1000.0 / 16)
_NPL = [_A * h * w for h, w in _LEVEL_HW]
_OFFS = [0]
for _n in _NPL:
    _OFFS.append(_OFFS[-1] + _n)
_TOTAL = _OFFS[-1]
_LANES = 128
_ROWS = 512
_PADDED = _ROWS * _LANES

_TOPK_LEVELS = [(_OFFS[i] // _LANES, _OFFS[i + 1] // _LANES, min(_PRE_NMS_TOP_N, _NPL[i]))
                for i in range(3)]
_ALL_KEEP_LO = _OFFS[3]
_ALL_KEEP_HI = _TOTAL

_LVL_CONST = np.zeros((_PADDED,), np.float32)
for _i, _n in enumerate(_NPL):
    _LVL_CONST[_OFFS[_i]:_OFFS[_i + 1]] = float(_i)
_LVL_CONST = _LVL_CONST.reshape(_ROWS, _LANES)

_RANK_CONST = np.full((_PADDED,), 2 ** 30, np.int32)
for _i, (_h, _w) in enumerate(_LEVEL_HW):
    _r = (np.arange(_h * _w, dtype=np.int64).reshape(1, _h * _w) * _A
          + np.arange(_A, dtype=np.int64).reshape(_A, 1)).reshape(-1)
    _RANK_CONST[_OFFS[_i]:_OFFS[_i + 1]] = (_r + _OFFS[_i]).astype(np.int32)
_RANK_CONST = _RANK_CONST.reshape(_ROWS, _LANES)


def _nms_body(sizes_ref, logits_ref, dx_ref, dy_ref, dw_ref, dh_ref,
              ax1_ref, ay1_ref, ax2_ref, ay2_ref, lvl_ref, rank_ref, out_ref):
    i = pl.program_id(0)
    img_h = sizes_ref[i, 0].astype(jnp.float32)
    img_w = sizes_ref[i, 1].astype(jnp.float32)

    logits = logits_ref[0]
    ax1, ay1 = ax1_ref[0], ay1_ref[0]
    ax2, ay2 = ax2_ref[0], ay2_ref[0]
    lvl = lvl_ref[...]
    rank = rank_ref[...]

    aw = ax2 - ax1
    ah = ay2 - ay1
    acx = ax1 + 0.5 * aw
    acy = ay1 + 0.5 * ah
    dwc = jnp.minimum(dw_ref[0], _BBOX_XFORM_CLIP)
    dhc = jnp.minimum(dh_ref[0], _BBOX_XFORM_CLIP)
    pcx = dx_ref[0] * aw + acx
    pcy = dy_ref[0] * ah + acy
    pw = jnp.exp(dwc) * aw
    ph = jnp.exp(dhc) * ah

    bx1 = jnp.clip(pcx - 0.5 * pw, 0.0, img_w)
    by1 = jnp.clip(pcy - 0.5 * ph, 0.0, img_h)
    bx2 = jnp.clip(pcx + 0.5 * pw, 0.0, img_w)
    by2 = jnp.clip(pcy + 0.5 * ph, 0.0, img_h)

    scores = 1.0 / (1.0 + jnp.exp(-logits))
    ws = bx2 - bx1
    hs = by2 - by1
    valid = (ws >= _MIN_SIZE) & (hs >= _MIN_SIZE) & (scores >= _SCORE_THRESH)

    bits = lax.bitcast_convert_type(logits, jnp.int32)
    keys = jnp.where(bits >= 0, bits, bits ^ jnp.int32(0x7FFFFFFF))

    def kth_threshold(r0, r1, k):
        seg = keys[r0:r1]

        def bit_step(b, t):
            cand = t + lax.shift_left(jnp.int32(1), jnp.int32(30) - b)
            cnt = jnp.sum((seg >= cand).astype(jnp.int32))
            return jnp.where(cnt >= k, cand, t)

        return lax.fori_loop(0, 31, bit_step, jnp.int32(-2147483648))

    rows = lax.broadcasted_iota(jnp.int32, (_ROWS, _LANES), 0)
    cols = lax.broadcasted_iota(jnp.int32, (_ROWS, _LANES), 1)
    lin = rows * _LANES + cols

    sel = (lin >= _ALL_KEEP_LO) & (lin < _ALL_KEEP_HI)
    for (r0, r1, k) in _TOPK_LEVELS:
        t = kth_threshold(r0, r1, k)
        sel = sel | ((keys >= t) & (lin >= r0 * _LANES) & (lin < r1 * _LANES))

    mask = sel & valid
    neg_inf = jnp.float32(-jnp.inf)

    mx = jnp.maximum(
        jnp.maximum(jnp.max(jnp.where(mask, bx1, 0.0)), jnp.max(jnp.where(mask, by1, 0.0))),
        jnp.maximum(jnp.max(jnp.where(mask, bx2, 0.0)), jnp.max(jnp.where(mask, by2, 0.0))))
    off = lvl * (mx + 1.0)
    sx1 = bx1 + off
    sy1 = by1 + off
    sx2 = bx2 + off
    sy2 = by2 + off
    areas = (sx2 - sx1) * (sy2 - sy1)
    cur0 = jnp.where(mask, scores, neg_inf)

    li = lax.broadcasted_iota(jnp.int32, (1, _LANES), 1)

    def nms_step(t, cur):
        m = jnp.max(cur)
        ok = m > neg_inf
        eq = cur == m
        lvlmin = jnp.min(jnp.where(eq, lvl, 1e9))
        eq = eq & (lvl == lvlmin)
        kmax = jnp.max(jnp.where(eq, keys, jnp.int32(-2147483648)))
        eq = eq & (keys == kmax)
        ridx = jnp.min(jnp.where(eq, rank, jnp.int32(2 ** 30)))
        pick = eq & (rank == ridx)
        onef = pick.astype(jnp.float32)
        psx1 = jnp.sum(onef * sx1)
        psy1 = jnp.sum(onef * sy1)
        psx2 = jnp.sum(onef * sx2)
        psy2 = jnp.sum(onef * sy2)
        pbx1 = jnp.sum(onef * bx1)
        pby1 = jnp.sum(onef * by1)
        pbx2 = jnp.sum(onef * bx2)
        pby2 = jnp.sum(onef * by2)
        psc = jnp.sum(onef * scores)
        parea = (psx2 - psx1) * (psy2 - psy1)

        xx1 = jnp.maximum(psx1, sx1)
        yy1 = jnp.maximum(psy1, sy1)
        xx2 = jnp.minimum(psx2, sx2)
        yy2 = jnp.minimum(psy2, sy2)
        inter = jnp.maximum(0.0, xx2 - xx1) * jnp.maximum(0.0, yy2 - yy1)
        iou = inter / (parea + areas - inter)
        new_cur = jnp.where(iou <= _NMS_THRESH, cur, neg_inf)
        new_cur = jnp.where(pick, neg_inf, new_cur)

        row = (jnp.where(li == 0, jnp.where(ok, pbx1, 0.0), 0.0)
               + jnp.where(li == 1, jnp.where(ok, pby1, 0.0), 0.0)
               + jnp.where(li == 2, jnp.where(ok, pbx2, 0.0), 0.0)
               + jnp.where(li == 3, jnp.where(ok, pby2, 0.0), 0.0)
               + jnp.where(li == 4, jnp.where(ok, psc, 0.0), 0.0))
        out_ref[0, pl.ds(t, 1), :] = row
        return new_cur

    lax.fori_loop(0, _POST_NMS_TOP_N, nms_step, cur0)


def _layout(obj_levels, del_levels, anchors):
    logit_parts, d_parts = [], ([], [], [], [])
    anc_parts = []
    for (h, w), o, d, off, n in zip(_LEVEL_HW, obj_levels, del_levels, _OFFS, _NPL):
        logit_parts.append(o.reshape(_N_IMGS, _A * h * w))
        d5 = d.reshape(_N_IMGS, _A, 4, h, w)
        for c in range(4):
            d_parts[c].append(d5[:, :, c].reshape(_N_IMGS, _A * h * w))
        a = anchors[:, off:off + n, :].reshape(_N_IMGS, h, w, _A, 4)
        anc_parts.append(jnp.transpose(a, (0, 3, 1, 2, 4)).reshape(_N_IMGS, n, 4))

    pad = _PADDED - _TOTAL

    def cat(parts, pad_val):
        x = jnp.concatenate(parts, axis=1)
        x = jnp.pad(x, ((0, 0), (0, pad)), constant_values=pad_val)
        return x.reshape(_N_IMGS, _ROWS, _LANES)

    logits = cat(logit_parts, -1e30)
    dx, dy, dw, dh = (cat(p, 0.0) for p in d_parts)
    anc = jnp.concatenate(anc_parts, axis=1)
    anc = jnp.pad(anc, ((0, 0), (0, pad), (0, 0)))
    ax1, ay1, ax2, ay2 = (anc[..., c].reshape(_N_IMGS, _ROWS, _LANES) for c in range(4))
    return logits, dx, dy, dw, dh, ax1, ay1, ax2, ay2


def kernel(obj_l0, obj_l1, obj_l2, obj_l3, obj_l4,
           del_l0, del_l1, del_l2, del_l3, del_l4,
           anchors, image_sizes):
    obj_levels = [obj_l0, obj_l1, obj_l2, obj_l3, obj_l4]
    del_levels = [del_l0, del_l1, del_l2, del_l3, del_l4]
    planes = _layout(obj_levels, del_levels, anchors)
    lvl = jnp.asarray(_LVL_CONST)
    rank = jnp.asarray(_RANK_CONST)

    img_spec = pl.BlockSpec((1, _ROWS, _LANES), lambda i: (i, 0, 0))
    const_spec = pl.BlockSpec((_ROWS, _LANES), lambda i: (0, 0))

    out = pl.pallas_call(
        _nms_body,
        grid=(_N_IMGS,),
        in_specs=[pl.BlockSpec(memory_space=pltpu.SMEM)] + [img_spec] * 9 + [const_spec] * 2,
        out_specs=pl.BlockSpec((1, _POST_NMS_TOP_N, _LANES), lambda i: (i, 0, 0)),
        out_shape=jax.ShapeDtypeStruct((_N_IMGS, _POST_NMS_TOP_N, _LANES), jnp.float32),
    )(image_sizes, *planes, lvl, rank)

    return out[:, :, 0:4], out[:, :, 4]


# speedup vs baseline: 13.0089x; 1.6043x over previous
import functools
import math

import jax
import jax.numpy as jnp
import numpy as np
from jax import lax
from jax.experimental import pallas as pl
from jax.experimental.pallas import tpu as pltpu
from jax.experimental.pallas import tpu_sc as plsc

_N_IMGS = 2
_A = 3
_LEVEL_HW = [(128, 128), (64, 64), (32, 32), (16, 16), (8, 8)]
_PRE_NMS_TOP_N = 1000
_POST_NMS_TOP_N = 1000
_NMS_THRESH = 0.7
_MIN_SIZE = 0.001
_BBOX_XFORM_CLIP = math.log(1000.0 / 16)
_NPL = [_A * h * w for h, w in _LEVEL_HW]
_OFFS = [0]
for _n in _NPL:
    _OFFS.append(_OFFS[-1] + _n)
_TOTAL = _OFFS[-1]
_LANES = 128
_ROWS = 512
_PADDED = _ROWS * _LANES
_DUMMY = _PADDED - 1

_NSUB = 16
_SLOTS_PER_SUB = 272
_CW = _NSUB * _SLOTS_PER_SUB
_CROWS = _CW // _LANES
_CHUNK = _PADDED // _NSUB
_IDXBUF = 2064

_TOPK_LEVELS = [(_OFFS[i] // _LANES, _OFFS[i + 1] // _LANES, min(_PRE_NMS_TOP_N, _NPL[i]))
                for i in range(3)]
_ALL_KEEP_LO = _OFFS[3]
_ALL_KEEP_HI = _TOTAL

_RANK_CONST = np.full((_PADDED,), 2 ** 30, np.int32)
for _i, (_h, _w) in enumerate(_LEVEL_HW):
    _r = (np.arange(_h * _w, dtype=np.int64).reshape(1, _h * _w) * _A
          + np.arange(_A, dtype=np.int64).reshape(_A, 1)).reshape(-1)
    _RANK_CONST[_OFFS[_i]:_OFFS[_i + 1]] = (_r + _OFFS[_i]).astype(np.int32)


def _prep_body(sizes_ref, logits_ref, dx_ref, dy_ref, dw_ref, dh_ref,
               ax1_ref, ay1_ref, ax2_ref, ay2_ref,
               sel_ref, bx1_ref, by1_ref, bx2_ref, by2_ref, mx_ref):
    i = pl.program_id(0)
    img_h = sizes_ref[i, 0].astype(jnp.float32)
    img_w = sizes_ref[i, 1].astype(jnp.float32)

    logits = logits_ref[0]
    ax1, ay1 = ax1_ref[0], ay1_ref[0]
    ax2, ay2 = ax2_ref[0], ay2_ref[0]

    aw = ax2 - ax1
    ah = ay2 - ay1
    acx = ax1 + 0.5 * aw
    acy = ay1 + 0.5 * ah
    dwc = jnp.minimum(dw_ref[0], _BBOX_XFORM_CLIP)
    dhc = jnp.minimum(dh_ref[0], _BBOX_XFORM_CLIP)
    pcx = dx_ref[0] * aw + acx
    pcy = dy_ref[0] * ah + acy
    pw = jnp.exp(dwc) * aw
    ph = jnp.exp(dhc) * ah

    bx1 = jnp.clip(pcx - 0.5 * pw, 0.0, img_w)
    by1 = jnp.clip(pcy - 0.5 * ph, 0.0, img_h)
    bx2 = jnp.clip(pcx + 0.5 * pw, 0.0, img_w)
    by2 = jnp.clip(pcy + 0.5 * ph, 0.0, img_h)

    valid = ((bx2 - bx1) >= _MIN_SIZE) & ((by2 - by1) >= _MIN_SIZE)

    bits = lax.bitcast_convert_type(logits, jnp.int32)
    keys = jnp.where(bits >= 0, bits, bits ^ jnp.int32(0x7FFFFFFF))

    def kth_threshold(r0, r1, k):
        seg = keys[r0:r1]

        def bit_step(b, t):
            cand = t + lax.shift_left(jnp.int32(1), jnp.int32(31) - b)
            cnt = jnp.sum((seg >= cand).astype(jnp.int32))
            return jnp.where(cnt >= k, cand, t)

        return lax.fori_loop(0, 32, bit_step, jnp.int32(-2147483648))

    rows = lax.broadcasted_iota(jnp.int32, (_ROWS, _LANES), 0)
    cols = lax.broadcasted_iota(jnp.int32, (_ROWS, _LANES), 1)
    lin = rows * _LANES + cols

    sel = (lin >= _ALL_KEEP_LO) & (lin < _ALL_KEEP_HI)
    for (r0, r1, k) in _TOPK_LEVELS:
        t = kth_threshold(r0, r1, k)
        sel = sel | ((keys >= t) & (lin >= r0 * _LANES) & (lin < r1 * _LANES))

    mask = sel & valid
    mx = jnp.maximum(
        jnp.maximum(jnp.max(jnp.where(mask, bx1, 0.0)), jnp.max(jnp.where(mask, by1, 0.0))),
        jnp.maximum(jnp.max(jnp.where(mask, bx2, 0.0)), jnp.max(jnp.where(mask, by2, 0.0))))

    sel_ref[0] = sel.astype(jnp.int32)
    bx1_ref[0] = bx1
    by1_ref[0] = by1
    bx2_ref[0] = bx2
    by2_ref[0] = by2
    mx_ref[0] = jnp.full((8, _LANES), mx, jnp.float32)


def _take16(v, idx):
    dnums = lax.GatherDimensionNumbers(offset_dims=(), collapsed_slice_dims=(0,),
                                       start_index_map=(0,))
    return lax.gather(v, idx[:, None], dnums, slice_sizes=(1,),
                      mode=lax.GatherScatterMode.PROMISE_IN_BOUNDS)


def _sc_compact_body(sel_h, bx1_h, by1_h, bx2_h, by2_h, log_h, rank_h,
                     idx_o, obx1, oby1, obx2, oby2, olog, orank,
                     selbuf, idxbuf, fillbuf, myidx, gixbuf, gbuf, grbuf, cnt, sem):
    img = lax.axis_index("c")
    sid = lax.axis_index("s")
    imgoff = img * _PADDED
    obase = img * _CW
    base0 = sid * _CHUNK
    slot0 = sid * _SLOTS_PER_SUB
    lanes = lax.iota(jnp.int32, 16)
    dummy16 = jnp.full((16,), _DUMMY, jnp.int32)

    @pl.when(sid == 0)
    def _():
        cnt[0] = 0

    def fill_body(j, c):
        fillbuf[pl.ds(j * 16, 16)] = dummy16
        return c
    lax.fori_loop(0, _SLOTS_PER_SUB // 16, fill_body, 0)
    pltpu.sync_copy(fillbuf, idx_o.at[pl.ds(pl.multiple_of(obase + slot0, 16), _SLOTS_PER_SUB)])

    pltpu.sync_copy(sel_h.at[pl.ds(pl.multiple_of(imgoff + base0, 16), _CHUNK)], selbuf)

    def prefix16(v):
        c = v
        for sh in (1, 2, 4, 8):
            shifted = _take16(c, jnp.maximum(lanes - sh, 0))
            c = c + jnp.where(lanes >= jnp.full((16,), sh, jnp.int32), shifted,
                              jnp.zeros((16,), jnp.int32))
        return c

    trash = jnp.full((16,), _IDXBUF - 1, jnp.int32)
    one16 = jnp.full((16,), 1, jnp.int32)

    def comp_body(j, ptr):
        s16 = selbuf[pl.ds(j * 16, 16)]
        gidx = (base0 + j * 16) + lanes
        pre = prefix16(s16)
        _, gsorted = plsc.sort_key_val(s16, gidx, descending=True)
        idxbuf[pl.ds(ptr, 16)] = gsorted
        return jnp.minimum(ptr + pre[15], _IDXBUF - 16)
    ptr = lax.fori_loop(0, _CHUNK // 16, comp_body, jnp.int32(0))

    idxbuf[pl.ds(ptr, 16)] = dummy16

    plsc.subcore_barrier()

    ngroups = (ptr + 15) // 16
    base = pl.multiple_of(plsc.fetch_and_add(cnt.at[0], ngroups * 16, subcore_id=0), 16)
    ngroups = jnp.minimum(ngroups, jnp.maximum((_CW - base) // 16, 0))

    def scat_body(j, c):
        pltpu.sync_copy(idxbuf.at[pl.ds(pl.multiple_of(j * 16, 16), 16)],
                        idx_o.at[pl.ds(pl.multiple_of(obase + base + j * 16, 16), 16)])
        return c
    lax.fori_loop(0, ngroups, scat_body, 0)

    plsc.subcore_barrier()

    pltpu.sync_copy(idx_o.at[pl.ds(pl.multiple_of(obase + slot0, 16), _SLOTS_PER_SUB)], myidx)

    def gix_body(j, c):
        gixbuf[pl.ds(j * 16, 16)] = myidx[pl.ds(j * 16, 16)] + imgoff
        return c
    lax.fori_loop(0, _SLOTS_PER_SUB // 16, gix_body, 0)

    chunks = [(0, 128), (128, 128), (256, 16)]
    for src, dst, buf, per_img in ((bx1_h, obx1, gbuf, True), (by1_h, oby1, gbuf, True),
                                   (bx2_h, obx2, gbuf, True), (by2_h, oby2, gbuf, True),
                                   (log_h, olog, gbuf, True), (rank_h, orank, grbuf, False)):
        ixref = gixbuf if per_img else myidx
        for (c0, cl) in chunks:
            pltpu.async_copy(src.at[ixref.at[pl.ds(c0, cl)]],
                             buf.at[pl.ds(c0, cl)], sem).wait()
        pltpu.sync_copy(buf, dst.at[pl.ds(pl.multiple_of(obase + slot0, 16), _SLOTS_PER_SUB)])


def _nms_body(mx_ref, idx_ref, bx1_ref, by1_ref, bx2_ref, by2_ref,
              log_ref, rank_ref, out_ref):
    mxv = jnp.max(mx_ref[0])
    idx = idx_ref[0]
    logits = log_ref[0]
    rank = rank_ref[0]
    bx1, by1 = bx1_ref[0], by1_ref[0]
    bx2, by2 = bx2_ref[0], by2_ref[0]

    scores = 1.0 / (1.0 + jnp.exp(-logits))
    bits = lax.bitcast_convert_type(logits, jnp.int32)
    keys = jnp.where(bits >= 0, bits, bits ^ jnp.int32(0x7FFFFFFF))
    lvl = ((idx >= _OFFS[1]).astype(jnp.float32) + (idx >= _OFFS[2]).astype(jnp.float32)
           + (idx >= _OFFS[3]).astype(jnp.float32) + (idx >= _OFFS[4]).astype(jnp.float32))
    valid = ((bx2 - bx1) >= _MIN_SIZE) & ((by2 - by1) >= _MIN_SIZE)

    neg_inf = jnp.float32(-jnp.inf)
    off = lvl * (mxv + 1.0)
    sx1 = bx1 + off
    sy1 = by1 + off
    sx2 = bx2 + off
    sy2 = by2 + off
    areas = (sx2 - sx1) * (sy2 - sy1)
    cur0 = jnp.where(valid, scores, neg_inf)

    li = lax.broadcasted_iota(jnp.int32, (1, _LANES), 1)

    def nms_step(t, cur):
        m = jnp.max(cur)
        ok = m > neg_inf
        eq = cur == m
        lvlmin = jnp.min(jnp.where(eq, lvl, 1e9))
        eq = eq & (lvl == lvlmin)
        kmax = jnp.max(jnp.where(eq, keys, jnp.int32(-2147483648)))
        eq = eq & (keys == kmax)
        ridx = jnp.min(jnp.where(eq, rank, jnp.int32(2 ** 30)))
        pick = eq & (rank == ridx)
        onef = pick.astype(jnp.float32)
        psx1 = jnp.sum(onef * sx1)
        psy1 = jnp.sum(onef * sy1)
        psx2 = jnp.sum(onef * sx2)
        psy2 = jnp.sum(onef * sy2)
        pbx1 = jnp.sum(onef * bx1)
        pby1 = jnp.sum(onef * by1)
        pbx2 = jnp.sum(onef * bx2)
        pby2 = jnp.sum(onef * by2)
        psc = jnp.sum(onef * scores)
        parea = (psx2 - psx1) * (psy2 - psy1)

        xx1 = jnp.maximum(psx1, sx1)
        yy1 = jnp.maximum(psy1, sy1)
        xx2 = jnp.minimum(psx2, sx2)
        yy2 = jnp.minimum(psy2, sy2)
        inter = jnp.maximum(0.0, xx2 - xx1) * jnp.maximum(0.0, yy2 - yy1)
        iou = inter / (parea + areas - inter)
        new_cur = jnp.where(iou <= _NMS_THRESH, cur, neg_inf)
        new_cur = jnp.where(pick, neg_inf, new_cur)

        row = (jnp.where(li == 0, jnp.where(ok, pbx1, 0.0), 0.0)
               + jnp.where(li == 1, jnp.where(ok, pby1, 0.0), 0.0)
               + jnp.where(li == 2, jnp.where(ok, pbx2, 0.0), 0.0)
               + jnp.where(li == 3, jnp.where(ok, pby2, 0.0), 0.0)
               + jnp.where(li == 4, jnp.where(ok, psc, 0.0), 0.0))
        out_ref[0, pl.ds(t, 1), :] = row
        return new_cur

    lax.fori_loop(0, _POST_NMS_TOP_N, nms_step, cur0)


def _layout(obj_levels, del_levels, anchors):
    logit_parts, d_parts = [], ([], [], [], [])
    anc_parts = []
    for (h, w), o, d, off, n in zip(_LEVEL_HW, obj_levels, del_levels, _OFFS, _NPL):
        logit_parts.append(o.reshape(_N_IMGS, _A * h * w))
        d5 = d.reshape(_N_IMGS, _A, 4, h, w)
        for c in range(4):
            d_parts[c].append(d5[:, :, c].reshape(_N_IMGS, _A * h * w))
        a = anchors[:, off:off + n, :].reshape(_N_IMGS, h, w, _A, 4)
        anc_parts.append(jnp.transpose(a, (0, 3, 1, 2, 4)).reshape(_N_IMGS, n, 4))

    pad = _PADDED - _TOTAL

    def cat(parts, pad_val):
        x = jnp.concatenate(parts, axis=1)
        x = jnp.pad(x, ((0, 0), (0, pad)), constant_values=pad_val)
        return x.reshape(_N_IMGS, _ROWS, _LANES)

    logits = cat(logit_parts, -1e30)
    dx, dy, dw, dh = (cat(p, 0.0) for p in d_parts)
    anc = jnp.concatenate(anc_parts, axis=1)
    anc = jnp.pad(anc, ((0, 0), (0, pad), (0, 0)))
    ax1, ay1, ax2, ay2 = (anc[..., c].reshape(_N_IMGS, _ROWS, _LANES) for c in range(4))
    return logits, dx, dy, dw, dh, ax1, ay1, ax2, ay2


def kernel(obj_l0, obj_l1, obj_l2, obj_l3, obj_l4,
           del_l0, del_l1, del_l2, del_l3, del_l4,
           anchors, image_sizes):
    obj_levels = [obj_l0, obj_l1, obj_l2, obj_l3, obj_l4]
    del_levels = [del_l0, del_l1, del_l2, del_l3, del_l4]
    planes = _layout(obj_levels, del_levels, anchors)
    logits3d = planes[0]

    img_spec = pl.BlockSpec((1, _ROWS, _LANES), lambda i: (i, 0, 0))

    plane3 = jax.ShapeDtypeStruct((_N_IMGS, _ROWS, _LANES), jnp.float32)
    sel, bx1p, by1p, bx2p, by2p, mxp = pl.pallas_call(
        _prep_body,
        grid=(_N_IMGS,),
        in_specs=[pl.BlockSpec(memory_space=pltpu.SMEM)] + [img_spec] * 9,
        out_specs=[img_spec] * 5 + [pl.BlockSpec((1, 8, _LANES), lambda i: (i, 0, 0))],
        out_shape=[jax.ShapeDtypeStruct((_N_IMGS, _ROWS, _LANES), jnp.int32),
                   plane3, plane3, plane3, plane3,
                   jax.ShapeDtypeStruct((_N_IMGS, 8, _LANES), jnp.float32)],
    )(image_sizes, *planes)

    flat = lambda x: x.reshape(_N_IMGS * _PADDED)
    rank = jnp.asarray(_RANK_CONST)

    cw_f32 = jax.ShapeDtypeStruct((_N_IMGS * _CW,), jnp.float32)
    cw_i32 = jax.ShapeDtypeStruct((_N_IMGS * _CW,), jnp.int32)
    sc_call = pl.kernel(
        _sc_compact_body,
        mesh=plsc.VectorSubcoreMesh(core_axis_name="c", subcore_axis_name="s"),
        compiler_params=pltpu.CompilerParams(needs_layout_passes=False),
        out_type=[cw_i32, cw_f32, cw_f32, cw_f32, cw_f32, cw_f32, cw_i32],
        scratch_types=[
            pltpu.VMEM((_CHUNK,), jnp.int32),
            pltpu.VMEM((_IDXBUF,), jnp.int32),
            pltpu.VMEM((_SLOTS_PER_SUB,), jnp.int32),
            pltpu.VMEM((_SLOTS_PER_SUB,), jnp.int32),
            pltpu.VMEM((_SLOTS_PER_SUB,), jnp.int32),
            pltpu.VMEM((_SLOTS_PER_SUB,), jnp.float32),
            pltpu.VMEM((_SLOTS_PER_SUB,), jnp.int32),
            pltpu.SMEM((1,), jnp.int32),
            pltpu.SemaphoreType.DMA,
        ],
    )
    gidx, gbx1, gby1, gbx2, gby2, glog, grank = sc_call(
        flat(sel), flat(bx1p), flat(by1p), flat(bx2p), flat(by2p),
        flat(logits3d), rank)

    c3 = lambda x: x.reshape(_N_IMGS, _CROWS, _LANES)
    cimg = pl.BlockSpec((1, _CROWS, _LANES), lambda i: (i, 0, 0))
    out = pl.pallas_call(
        _nms_body,
        grid=(_N_IMGS,),
        in_specs=[pl.BlockSpec((1, 8, _LANES), lambda i: (i, 0, 0))] + [cimg] * 7,
        out_specs=pl.BlockSpec((1, _POST_NMS_TOP_N, _LANES), lambda i: (i, 0, 0)),
        out_shape=jax.ShapeDtypeStruct((_N_IMGS, _POST_NMS_TOP_N, _LANES), jnp.float32),
    )(mxp, c3(gidx), c3(gbx1), c3(gby1), c3(gbx2), c3(gby2), c3(glog), c3(grank))

    return out[:, :, 0:4], out[:, :, 4]


# speedup vs baseline: 13.1094x; 1.0077x over previous
import functools
import math

import jax
import jax.numpy as jnp
import numpy as np
from jax import lax
from jax.experimental import pallas as pl
from jax.experimental.pallas import tpu as pltpu
from jax.experimental.pallas import tpu_sc as plsc

_N_IMGS = 2
_A = 3
_LEVEL_HW = [(128, 128), (64, 64), (32, 32), (16, 16), (8, 8)]
_PRE_NMS_TOP_N = 1000
_POST_NMS_TOP_N = 1000
_NMS_THRESH = 0.7
_MIN_SIZE = 0.001
_BBOX_XFORM_CLIP = math.log(1000.0 / 16)
_NPL = [_A * h * w for h, w in _LEVEL_HW]
_OFFS = [0]
for _n in _NPL:
    _OFFS.append(_OFFS[-1] + _n)
_TOTAL = _OFFS[-1]
_LANES = 128
_ROWS = 512
_PADDED = _ROWS * _LANES
_DUMMY = _PADDED - 1

_NSUB = 16
_SLOTS_PER_SUB = 272
_CW = _NSUB * _SLOTS_PER_SUB
_CROWS = _CW // _LANES
_CHUNK = _PADDED // _NSUB
_IDXBUF = 2064

_TOPK_LEVELS = [(_OFFS[i] // _LANES, _OFFS[i + 1] // _LANES, min(_PRE_NMS_TOP_N, _NPL[i]))
                for i in range(3)]
_ALL_KEEP_LO = _OFFS[3]
_ALL_KEEP_HI = _TOTAL

_RANK_CONST = np.full((_PADDED,), 2 ** 30, np.int32)
for _i, (_h, _w) in enumerate(_LEVEL_HW):
    _r = (np.arange(_h * _w, dtype=np.int64).reshape(1, _h * _w) * _A
          + np.arange(_A, dtype=np.int64).reshape(_A, 1)).reshape(-1)
    _RANK_CONST[_OFFS[_i]:_OFFS[_i + 1]] = (_r + _OFFS[_i]).astype(np.int32)


def _prep_body(sizes_ref, logits_ref, dx_ref, dy_ref, dw_ref, dh_ref,
               ax1_ref, ay1_ref, ax2_ref, ay2_ref,
               sel_ref, bx1_ref, by1_ref, bx2_ref, by2_ref, mx_ref):
    i = pl.program_id(0)
    img_h = sizes_ref[i, 0].astype(jnp.float32)
    img_w = sizes_ref[i, 1].astype(jnp.float32)

    logits = logits_ref[0]
    ax1, ay1 = ax1_ref[0], ay1_ref[0]
    ax2, ay2 = ax2_ref[0], ay2_ref[0]

    aw = ax2 - ax1
    ah = ay2 - ay1
    acx = ax1 + 0.5 * aw
    acy = ay1 + 0.5 * ah
    dwc = jnp.minimum(dw_ref[0], _BBOX_XFORM_CLIP)
    dhc = jnp.minimum(dh_ref[0], _BBOX_XFORM_CLIP)
    pcx = dx_ref[0] * aw + acx
    pcy = dy_ref[0] * ah + acy
    pw = jnp.exp(dwc) * aw
    ph = jnp.exp(dhc) * ah

    bx1 = jnp.clip(pcx - 0.5 * pw, 0.0, img_w)
    by1 = jnp.clip(pcy - 0.5 * ph, 0.0, img_h)
    bx2 = jnp.clip(pcx + 0.5 * pw, 0.0, img_w)
    by2 = jnp.clip(pcy + 0.5 * ph, 0.0, img_h)

    valid = ((bx2 - bx1) >= _MIN_SIZE) & ((by2 - by1) >= _MIN_SIZE)

    bits = lax.bitcast_convert_type(logits, jnp.int32)
    keys = jnp.where(bits >= 0, bits, bits ^ jnp.int32(0x7FFFFFFF))

    def kth_threshold(r0, r1, k):
        seg = keys[r0:r1]

        def bit_step(b, t):
            cand = t + lax.shift_left(jnp.int32(1), jnp.int32(31) - b)
            cnt = jnp.sum((seg >= cand).astype(jnp.int32))
            return jnp.where(cnt >= k, cand, t)

        return lax.fori_loop(0, 32, bit_step, jnp.int32(-2147483648))

    rows = lax.broadcasted_iota(jnp.int32, (_ROWS, _LANES), 0)
    cols = lax.broadcasted_iota(jnp.int32, (_ROWS, _LANES), 1)
    lin = rows * _LANES + cols

    sel = (lin >= _ALL_KEEP_LO) & (lin < _ALL_KEEP_HI)
    for (r0, r1, k) in _TOPK_LEVELS:
        t = kth_threshold(r0, r1, k)
        sel = sel | ((keys >= t) & (lin >= r0 * _LANES) & (lin < r1 * _LANES))

    mask = sel & valid
    mx = jnp.maximum(
        jnp.maximum(jnp.max(jnp.where(mask, bx1, 0.0)), jnp.max(jnp.where(mask, by1, 0.0))),
        jnp.maximum(jnp.max(jnp.where(mask, bx2, 0.0)), jnp.max(jnp.where(mask, by2, 0.0))))

    sel_ref[0] = sel.astype(jnp.int32)
    bx1_ref[0] = bx1
    by1_ref[0] = by1
    bx2_ref[0] = bx2
    by2_ref[0] = by2
    mx_ref[0] = jnp.full((8, _LANES), mx, jnp.float32)


def _take16(v, idx):
    dnums = lax.GatherDimensionNumbers(offset_dims=(), collapsed_slice_dims=(0,),
                                       start_index_map=(0,))
    return lax.gather(v, idx[:, None], dnums, slice_sizes=(1,),
                      mode=lax.GatherScatterMode.PROMISE_IN_BOUNDS)


def _sc_compact_body(sel_h, bx1_h, by1_h, bx2_h, by2_h, log_h, rank_h,
                     idx_o, obx1, oby1, obx2, oby2, olog, orank,
                     selbuf, idxbuf, fillbuf, myidx, gixbuf, gbuf, grbuf, cnt, sem):
    img = lax.axis_index("c")
    sid = lax.axis_index("s")
    imgoff = img * _PADDED
    obase = img * _CW
    base0 = sid * _CHUNK
    slot0 = sid * _SLOTS_PER_SUB
    lanes = lax.iota(jnp.int32, 16)
    dummy16 = jnp.full((16,), _DUMMY, jnp.int32)

    @pl.when(sid == 0)
    def _():
        cnt[0] = 0

    def fill_body(j, c):
        fillbuf[pl.ds(j * 16, 16)] = dummy16
        return c
    lax.fori_loop(0, _SLOTS_PER_SUB // 16, fill_body, 0)
    pltpu.sync_copy(fillbuf, idx_o.at[pl.ds(pl.multiple_of(obase + slot0, 16), _SLOTS_PER_SUB)])

    pltpu.sync_copy(sel_h.at[pl.ds(pl.multiple_of(imgoff + base0, 16), _CHUNK)], selbuf)

    def prefix16(v):
        c = v
        for sh in (1, 2, 4, 8):
            shifted = _take16(c, jnp.maximum(lanes - sh, 0))
            c = c + jnp.where(lanes >= jnp.full((16,), sh, jnp.int32), shifted,
                              jnp.zeros((16,), jnp.int32))
        return c

    trash = jnp.full((16,), _IDXBUF - 1, jnp.int32)
    one16 = jnp.full((16,), 1, jnp.int32)

    def comp_body(j, ptr):
        s16 = selbuf[pl.ds(j * 16, 16)]
        gidx = (base0 + j * 16) + lanes
        pre = prefix16(s16)
        _, gsorted = plsc.sort_key_val(s16, gidx, descending=True)
        idxbuf[pl.ds(ptr, 16)] = gsorted
        return jnp.minimum(ptr + pre[15], _IDXBUF - 16)
    ptr = lax.fori_loop(0, _CHUNK // 16, comp_body, jnp.int32(0))

    idxbuf[pl.ds(ptr, 16)] = dummy16

    plsc.subcore_barrier()

    ngroups = (ptr + 15) // 16
    base = pl.multiple_of(plsc.fetch_and_add(cnt.at[0], ngroups * 16, subcore_id=0), 16)
    ngroups = jnp.minimum(ngroups, jnp.maximum((_CW - base) // 16, 0))

    def scat_body(j, c):
        pltpu.sync_copy(idxbuf.at[pl.ds(pl.multiple_of(j * 16, 16), 16)],
                        idx_o.at[pl.ds(pl.multiple_of(obase + base + j * 16, 16), 16)])
        return c
    lax.fori_loop(0, ngroups, scat_body, 0)

    plsc.subcore_barrier()

    pltpu.sync_copy(idx_o.at[pl.ds(pl.multiple_of(obase + slot0, 16), _SLOTS_PER_SUB)], myidx)

    def gix_body(j, c):
        gixbuf[pl.ds(j * 16, 16)] = myidx[pl.ds(j * 16, 16)] + imgoff
        return c
    lax.fori_loop(0, _SLOTS_PER_SUB // 16, gix_body, 0)

    chunks = [(0, 128), (128, 128), (256, 16)]
    planes_io = ((bx1_h, obx1, 0, True), (by1_h, oby1, 1, True), (bx2_h, obx2, 2, True),
                 (by2_h, oby2, 3, True), (log_h, olog, 4, True))
    handles = []
    for src, _, slot, per_img in planes_io:
        ixref = gixbuf if per_img else myidx
        for (c0, cl) in chunks:
            handles.append(pltpu.async_copy(
                src.at[ixref.at[pl.ds(c0, cl)]],
                gbuf.at[pl.ds(slot * _SLOTS_PER_SUB + c0, cl)], sem))
    for (c0, cl) in chunks:
        handles.append(pltpu.async_copy(rank_h.at[myidx.at[pl.ds(c0, cl)]],
                                        grbuf.at[pl.ds(c0, cl)], sem))
    for h in handles:
        h.wait()
    for _, dst, slot, _ in planes_io:
        pltpu.sync_copy(gbuf.at[pl.ds(slot * _SLOTS_PER_SUB, _SLOTS_PER_SUB)],
                        dst.at[pl.ds(pl.multiple_of(obase + slot0, 16), _SLOTS_PER_SUB)])
    pltpu.sync_copy(grbuf, orank.at[pl.ds(pl.multiple_of(obase + slot0, 16), _SLOTS_PER_SUB)])


def _nms_body(mx_ref, idx_ref, bx1_ref, by1_ref, bx2_ref, by2_ref,
              log_ref, rank_ref, out_ref):
    mxv = jnp.max(mx_ref[0])
    idx = idx_ref[0]
    logits = log_ref[0]
    rank = rank_ref[0]
    bx1, by1 = bx1_ref[0], by1_ref[0]
    bx2, by2 = bx2_ref[0], by2_ref[0]

    scores = 1.0 / (1.0 + jnp.exp(-logits))
    bits = lax.bitcast_convert_type(logits, jnp.int32)
    keys = jnp.where(bits >= 0, bits, bits ^ jnp.int32(0x7FFFFFFF))
    lvl = ((idx >= _OFFS[1]).astype(jnp.float32) + (idx >= _OFFS[2]).astype(jnp.float32)
           + (idx >= _OFFS[3]).astype(jnp.float32) + (idx >= _OFFS[4]).astype(jnp.float32))
    valid = ((bx2 - bx1) >= _MIN_SIZE) & ((by2 - by1) >= _MIN_SIZE)

    neg_inf = jnp.float32(-jnp.inf)
    off = lvl * (mxv + 1.0)
    sx1 = bx1 + off
    sy1 = by1 + off
    sx2 = bx2 + off
    sy2 = by2 + off
    areas = (sx2 - sx1) * (sy2 - sy1)
    cur0 = jnp.where(valid, scores, neg_inf)

    li = lax.broadcasted_iota(jnp.int32, (1, _LANES), 1)

    def nms_step(t, cur):
        m = jnp.max(cur)
        ok = m > neg_inf
        eq = cur == m
        lvlmin = jnp.min(jnp.where(eq, lvl, 1e9))
        eq = eq & (lvl == lvlmin)
        kmax = jnp.max(jnp.where(eq, keys, jnp.int32(-2147483648)))
        eq = eq & (keys == kmax)
        ridx = jnp.min(jnp.where(eq, rank, jnp.int32(2 ** 30)))
        pick = eq & (rank == ridx)
        onef = pick.astype(jnp.float32)
        psx1 = jnp.sum(onef * sx1)
        psy1 = jnp.sum(onef * sy1)
        psx2 = jnp.sum(onef * sx2)
        psy2 = jnp.sum(onef * sy2)
        pbx1 = jnp.sum(onef * bx1)
        pby1 = jnp.sum(onef * by1)
        pbx2 = jnp.sum(onef * bx2)
        pby2 = jnp.sum(onef * by2)
        psc = jnp.sum(onef * scores)
        parea = (psx2 - psx1) * (psy2 - psy1)

        xx1 = jnp.maximum(psx1, sx1)
        yy1 = jnp.maximum(psy1, sy1)
        xx2 = jnp.minimum(psx2, sx2)
        yy2 = jnp.minimum(psy2, sy2)
        inter = jnp.maximum(0.0, xx2 - xx1) * jnp.maximum(0.0, yy2 - yy1)
        iou = inter / (parea + areas - inter)
        new_cur = jnp.where(iou <= _NMS_THRESH, cur, neg_inf)
        new_cur = jnp.where(pick, neg_inf, new_cur)

        row = (jnp.where(li == 0, jnp.where(ok, pbx1, 0.0), 0.0)
               + jnp.where(li == 1, jnp.where(ok, pby1, 0.0), 0.0)
               + jnp.where(li == 2, jnp.where(ok, pbx2, 0.0), 0.0)
               + jnp.where(li == 3, jnp.where(ok, pby2, 0.0), 0.0)
               + jnp.where(li == 4, jnp.where(ok, psc, 0.0), 0.0))
        out_ref[0, pl.ds(t, 1), :] = row
        return new_cur

    lax.fori_loop(0, _POST_NMS_TOP_N, nms_step, cur0)


def _layout(obj_levels, del_levels, anchors):
    logit_parts, d_parts = [], ([], [], [], [])
    anc_parts = []
    for (h, w), o, d, off, n in zip(_LEVEL_HW, obj_levels, del_levels, _OFFS, _NPL):
        logit_parts.append(o.reshape(_N_IMGS, _A * h * w))
        d5 = d.reshape(_N_IMGS, _A, 4, h, w)
        for c in range(4):
            d_parts[c].append(d5[:, :, c].reshape(_N_IMGS, _A * h * w))
        a = anchors[:, off:off + n, :].reshape(_N_IMGS, h, w, _A, 4)
        anc_parts.append(jnp.transpose(a, (0, 3, 1, 2, 4)).reshape(_N_IMGS, n, 4))

    pad = _PADDED - _TOTAL

    def cat(parts, pad_val):
        x = jnp.concatenate(parts, axis=1)
        x = jnp.pad(x, ((0, 0), (0, pad)), constant_values=pad_val)
        return x.reshape(_N_IMGS, _ROWS, _LANES)

    logits = cat(logit_parts, -1e30)
    dx, dy, dw, dh = (cat(p, 0.0) for p in d_parts)
    anc = jnp.concatenate(anc_parts, axis=1)
    anc = jnp.pad(anc, ((0, 0), (0, pad), (0, 0)))
    ax1, ay1, ax2, ay2 = (anc[..., c].reshape(_N_IMGS, _ROWS, _LANES) for c in range(4))
    return logits, dx, dy, dw, dh, ax1, ay1, ax2, ay2


def kernel(obj_l0, obj_l1, obj_l2, obj_l3, obj_l4,
           del_l0, del_l1, del_l2, del_l3, del_l4,
           anchors, image_sizes):
    obj_levels = [obj_l0, obj_l1, obj_l2, obj_l3, obj_l4]
    del_levels = [del_l0, del_l1, del_l2, del_l3, del_l4]
    planes = _layout(obj_levels, del_levels, anchors)
    logits3d = planes[0]

    img_spec = pl.BlockSpec((1, _ROWS, _LANES), lambda i: (i, 0, 0))

    plane3 = jax.ShapeDtypeStruct((_N_IMGS, _ROWS, _LANES), jnp.float32)
    sel, bx1p, by1p, bx2p, by2p, mxp = pl.pallas_call(
        _prep_body,
        grid=(_N_IMGS,),
        in_specs=[pl.BlockSpec(memory_space=pltpu.SMEM)] + [img_spec] * 9,
        out_specs=[img_spec] * 5 + [pl.BlockSpec((1, 8, _LANES), lambda i: (i, 0, 0))],
        out_shape=[jax.ShapeDtypeStruct((_N_IMGS, _ROWS, _LANES), jnp.int32),
                   plane3, plane3, plane3, plane3,
                   jax.ShapeDtypeStruct((_N_IMGS, 8, _LANES), jnp.float32)],
    )(image_sizes, *planes)

    flat = lambda x: x.reshape(_N_IMGS * _PADDED)
    rank = jnp.asarray(_RANK_CONST)

    cw_f32 = jax.ShapeDtypeStruct((_N_IMGS * _CW,), jnp.float32)
    cw_i32 = jax.ShapeDtypeStruct((_N_IMGS * _CW,), jnp.int32)
    sc_call = pl.kernel(
        _sc_compact_body,
        mesh=plsc.VectorSubcoreMesh(core_axis_name="c", subcore_axis_name="s"),
        compiler_params=pltpu.CompilerParams(needs_layout_passes=False),
        out_type=[cw_i32, cw_f32, cw_f32, cw_f32, cw_f32, cw_f32, cw_i32],
        scratch_types=[
            pltpu.VMEM((_CHUNK,), jnp.int32),
            pltpu.VMEM((_IDXBUF,), jnp.int32),
            pltpu.VMEM((_SLOTS_PER_SUB,), jnp.int32),
            pltpu.VMEM((_SLOTS_PER_SUB,), jnp.int32),
            pltpu.VMEM((_SLOTS_PER_SUB,), jnp.int32),
            pltpu.VMEM((5 * _SLOTS_PER_SUB,), jnp.float32),
            pltpu.VMEM((_SLOTS_PER_SUB,), jnp.int32),
            pltpu.SMEM((1,), jnp.int32),
            pltpu.SemaphoreType.DMA,
        ],
    )
    gidx, gbx1, gby1, gbx2, gby2, glog, grank = sc_call(
        flat(sel), flat(bx1p), flat(by1p), flat(bx2p), flat(by2p),
        flat(logits3d), rank)

    c3 = lambda x: x.reshape(_N_IMGS, _CROWS, _LANES)
    cimg = pl.BlockSpec((1, _CROWS, _LANES), lambda i: (i, 0, 0))
    out = pl.pallas_call(
        _nms_body,
        grid=(_N_IMGS,),
        in_specs=[pl.BlockSpec((1, 8, _LANES), lambda i: (i, 0, 0))] + [cimg] * 7,
        out_specs=pl.BlockSpec((1, _POST_NMS_TOP_N, _LANES), lambda i: (i, 0, 0)),
        out_shape=jax.ShapeDtypeStruct((_N_IMGS, _POST_NMS_TOP_N, _LANES), jnp.float32),
    )(mxp, c3(gidx), c3(gbx1), c3(gby1), c3(gbx2), c3(gby2), c3(glog), c3(grank))

    return out[:, :, 0:4], out[:, :, 4]


# speedup vs baseline: 13.1147x; 1.0004x over previous
import math

import jax
import jax.numpy as jnp
import numpy as np
from jax import lax
from jax.experimental import pallas as pl
from jax.experimental.pallas import tpu as pltpu
from jax.experimental.pallas import tpu_sc as plsc

_N_IMGS = 2
_A = 3
_LEVEL_HW = [(128, 128), (64, 64), (32, 32), (16, 16), (8, 8)]
_PRE_NMS_TOP_N = 1000
_POST_NMS_TOP_N = 1000
_NMS_THRESH = 0.7
_MIN_SIZE = 0.001
_BBOX_XFORM_CLIP = math.log(1000.0 / 16)
_NPL = [_A * h * w for h, w in _LEVEL_HW]
_OFFS = [0]
for _n in _NPL:
    _OFFS.append(_OFFS[-1] + _n)
_TOTAL = _OFFS[-1]
_LANES = 128
_ROWS = 512
_PADDED = _ROWS * _LANES
_DUMMY = _PADDED - 1

_NSUB = 16
_SLOTS_PER_SUB = 272
_CW = _NSUB * _SLOTS_PER_SUB
_CROWS = _CW // _LANES
_CHUNK = _PADDED // _NSUB
_IDXBUF = 2064

_TOPK_LEVELS = [(_OFFS[i] // _LANES, _OFFS[i + 1] // _LANES, min(_PRE_NMS_TOP_N, _NPL[i]))
                for i in range(3)]
_ALL_KEEP_LO = _OFFS[3]
_ALL_KEEP_HI = _TOTAL

_RANK_CONST = np.full((_PADDED,), 2 ** 30, np.int32)
for _i, (_h, _w) in enumerate(_LEVEL_HW):
    _r = (np.arange(_h * _w, dtype=np.int64).reshape(1, _h * _w) * _A
          + np.arange(_A, dtype=np.int64).reshape(_A, 1)).reshape(-1)
    _RANK_CONST[_OFFS[_i]:_OFFS[_i + 1]] = (_r + _OFFS[_i]).astype(np.int32)


def _prep_body(sizes_ref, logits_ref, dx_ref, dy_ref, dw_ref, dh_ref,
               ax1_ref, ay1_ref, ax2_ref, ay2_ref,
               sel_ref, bx1_ref, by1_ref, bx2_ref, by2_ref, mx_ref):
    i = pl.program_id(0)
    img_h = sizes_ref[i, 0].astype(jnp.float32)
    img_w = sizes_ref[i, 1].astype(jnp.float32)

    logits = logits_ref[0]
    ax1, ay1 = ax1_ref[0], ay1_ref[0]
    ax2, ay2 = ax2_ref[0], ay2_ref[0]

    aw = ax2 - ax1
    ah = ay2 - ay1
    acx = ax1 + 0.5 * aw
    acy = ay1 + 0.5 * ah
    dwc = jnp.minimum(dw_ref[0], _BBOX_XFORM_CLIP)
    dhc = jnp.minimum(dh_ref[0], _BBOX_XFORM_CLIP)
    pcx = dx_ref[0] * aw + acx
    pcy = dy_ref[0] * ah + acy
    pw = jnp.exp(dwc) * aw
    ph = jnp.exp(dhc) * ah

    bx1 = jnp.clip(pcx - 0.5 * pw, 0.0, img_w)
    by1 = jnp.clip(pcy - 0.5 * ph, 0.0, img_h)
    bx2 = jnp.clip(pcx + 0.5 * pw, 0.0, img_w)
    by2 = jnp.clip(pcy + 0.5 * ph, 0.0, img_h)

    valid = ((bx2 - bx1) >= _MIN_SIZE) & ((by2 - by1) >= _MIN_SIZE)

    bits = lax.bitcast_convert_type(logits, jnp.int32)
    keys = jnp.where(bits >= 0, bits, bits ^ jnp.int32(0x7FFFFFFF))

    def kth_threshold(r0, r1, k):
        seg = keys[r0:r1]

        def bit_step(b, t):
            cand = t + lax.shift_left(jnp.int32(1), jnp.int32(31) - b)
            cnt = jnp.sum((seg >= cand).astype(jnp.int32))
            return jnp.where(cnt >= k, cand, t)

        return lax.fori_loop(0, 32, bit_step, jnp.int32(-2147483648))

    rows = lax.broadcasted_iota(jnp.int32, (_ROWS, _LANES), 0)
    cols = lax.broadcasted_iota(jnp.int32, (_ROWS, _LANES), 1)
    lin = rows * _LANES + cols

    sel = (lin >= _ALL_KEEP_LO) & (lin < _ALL_KEEP_HI)
    for (r0, r1, k) in _TOPK_LEVELS:
        t = kth_threshold(r0, r1, k)
        sel = sel | ((keys >= t) & (lin >= r0 * _LANES) & (lin < r1 * _LANES))

    mask = sel & valid
    mx = jnp.maximum(
        jnp.maximum(jnp.max(jnp.where(mask, bx1, 0.0)), jnp.max(jnp.where(mask, by1, 0.0))),
        jnp.maximum(jnp.max(jnp.where(mask, bx2, 0.0)), jnp.max(jnp.where(mask, by2, 0.0))))

    sel_ref[0] = sel.astype(jnp.int32)
    bx1_ref[0] = bx1
    by1_ref[0] = by1
    bx2_ref[0] = bx2
    by2_ref[0] = by2
    mx_ref[0] = jnp.full((8, _LANES), mx, jnp.float32)


def _take16(v, idx):
    dnums = lax.GatherDimensionNumbers(offset_dims=(), collapsed_slice_dims=(0,),
                                       start_index_map=(0,))
    return lax.gather(v, idx[:, None], dnums, slice_sizes=(1,),
                      mode=lax.GatherScatterMode.PROMISE_IN_BOUNDS)


def _sc_compact_body(sel_h, bx1_h, by1_h, bx2_h, by2_h, log_h, rank_h,
                     idx_o, obx1, oby1, obx2, oby2, olog, orank,
                     selbuf, idxbuf, fillbuf, myidx, gixbuf, gbuf, grbuf, cnt, sem):
    img = lax.axis_index("c")
    sid = lax.axis_index("s")
    imgoff = img * _PADDED
    obase = img * _CW
    base0 = sid * _CHUNK
    slot0 = sid * _SLOTS_PER_SUB
    lanes = lax.iota(jnp.int32, 16)
    dummy16 = jnp.full((16,), _DUMMY, jnp.int32)

    @pl.when(sid == 0)
    def _():
        cnt[0] = 0

    def fill_body(j, c):
        fillbuf[pl.ds(j * 16, 16)] = dummy16
        return c
    lax.fori_loop(0, _SLOTS_PER_SUB // 16, fill_body, 0)
    pltpu.sync_copy(fillbuf, idx_o.at[pl.ds(pl.multiple_of(obase + slot0, 16), _SLOTS_PER_SUB)])

    pltpu.sync_copy(sel_h.at[pl.ds(pl.multiple_of(imgoff + base0, 16), _CHUNK)], selbuf)

    def prefix16(v):
        c = v
        for sh in (1, 2, 4, 8):
            shifted = _take16(c, jnp.maximum(lanes - sh, 0))
            c = c + jnp.where(lanes >= jnp.full((16,), sh, jnp.int32), shifted,
                              jnp.zeros((16,), jnp.int32))
        return c

    def comp_body(j, ptr):
        s16 = selbuf[pl.ds(j * 16, 16)]
        gidx = (base0 + j * 16) + lanes
        pre = prefix16(s16)
        _, gsorted = plsc.sort_key_val(s16, gidx, descending=True)
        idxbuf[pl.ds(ptr, 16)] = gsorted
        return jnp.minimum(ptr + pre[15], _IDXBUF - 16)
    ptr = lax.fori_loop(0, _CHUNK // 16, comp_body, jnp.int32(0))

    idxbuf[pl.ds(ptr, 16)] = dummy16

    plsc.subcore_barrier()

    ngroups = (ptr + 15) // 16
    base = pl.multiple_of(plsc.fetch_and_add(cnt.at[0], ngroups * 16, subcore_id=0), 16)
    ngroups = jnp.minimum(ngroups, jnp.maximum((_CW - base) // 16, 0))

    def scat_body(j, c):
        pltpu.sync_copy(idxbuf.at[pl.ds(pl.multiple_of(j * 16, 16), 16)],
                        idx_o.at[pl.ds(pl.multiple_of(obase + base + j * 16, 16), 16)])
        return c
    lax.fori_loop(0, ngroups, scat_body, 0)

    plsc.subcore_barrier()

    pltpu.sync_copy(idx_o.at[pl.ds(pl.multiple_of(obase + slot0, 16), _SLOTS_PER_SUB)], myidx)

    def gix_body(j, c):
        gixbuf[pl.ds(j * 16, 16)] = myidx[pl.ds(j * 16, 16)] + imgoff
        return c
    lax.fori_loop(0, _SLOTS_PER_SUB // 16, gix_body, 0)

    chunks = [(0, 128), (128, 128), (256, 16)]
    planes_io = ((bx1_h, obx1, 0, True), (by1_h, oby1, 1, True), (bx2_h, obx2, 2, True),
                 (by2_h, oby2, 3, True), (log_h, olog, 4, True))
    handles = []
    for src, _, slot, per_img in planes_io:
        ixref = gixbuf if per_img else myidx
        for (c0, cl) in chunks:
            handles.append(pltpu.async_copy(
                src.at[ixref.at[pl.ds(c0, cl)]],
                gbuf.at[pl.ds(slot * _SLOTS_PER_SUB + c0, cl)], sem))
    for (c0, cl) in chunks:
        handles.append(pltpu.async_copy(rank_h.at[myidx.at[pl.ds(c0, cl)]],
                                        grbuf.at[pl.ds(c0, cl)], sem))
    for h in handles:
        h.wait()
    for _, dst, slot, _ in planes_io:
        pltpu.sync_copy(gbuf.at[pl.ds(slot * _SLOTS_PER_SUB, _SLOTS_PER_SUB)],
                        dst.at[pl.ds(pl.multiple_of(obase + slot0, 16), _SLOTS_PER_SUB)])
    pltpu.sync_copy(grbuf, orank.at[pl.ds(pl.multiple_of(obase + slot0, 16), _SLOTS_PER_SUB)])


def _nms_body(mx_ref, idx_ref, bx1_ref, by1_ref, bx2_ref, by2_ref,
              log_ref, rank_ref, out_ref):
    mxv = jnp.max(mx_ref[0])
    idx = idx_ref[0]
    logits = log_ref[0]
    rank = rank_ref[0]
    bx1, by1 = bx1_ref[0], by1_ref[0]
    bx2, by2 = bx2_ref[0], by2_ref[0]

    scores = 1.0 / (1.0 + jnp.exp(-logits))
    bits = lax.bitcast_convert_type(logits, jnp.int32)
    keys = jnp.where(bits >= 0, bits, bits ^ jnp.int32(0x7FFFFFFF))
    lvl = ((idx >= _OFFS[1]).astype(jnp.float32) + (idx >= _OFFS[2]).astype(jnp.float32)
           + (idx >= _OFFS[3]).astype(jnp.float32) + (idx >= _OFFS[4]).astype(jnp.float32))
    valid = ((bx2 - bx1) >= _MIN_SIZE) & ((by2 - by1) >= _MIN_SIZE)

    neg_inf = jnp.float32(-jnp.inf)
    off = lvl * (mxv + 1.0)
    sx1 = bx1 + off
    sy1 = by1 + off
    sx2 = bx2 + off
    sy2 = by2 + off
    areas = (sx2 - sx1) * (sy2 - sy1)
    cur0 = jnp.where(valid, scores, neg_inf)

    li = lax.broadcasted_iota(jnp.int32, (1, _LANES), 1)

    def nms_step(t, cur):
        m = jnp.max(cur)
        ok = m > neg_inf
        eq = cur == m
        lvlmin = jnp.min(jnp.where(eq, lvl, 1e9))
        eq = eq & (lvl == lvlmin)
        kmax = jnp.max(jnp.where(eq, keys, jnp.int32(-2147483648)))
        eq = eq & (keys == kmax)
        ridx = jnp.min(jnp.where(eq, rank, jnp.int32(2 ** 30)))
        pick = eq & (rank == ridx)
        onef = pick.astype(jnp.float32)
        psx1 = jnp.sum(onef * sx1)
        psy1 = jnp.sum(onef * sy1)
        psx2 = jnp.sum(onef * sx2)
        psy2 = jnp.sum(onef * sy2)
        pbx1 = jnp.sum(onef * bx1)
        pby1 = jnp.sum(onef * by1)
        pbx2 = jnp.sum(onef * bx2)
        pby2 = jnp.sum(onef * by2)
        psc = jnp.sum(onef * scores)
        parea = (psx2 - psx1) * (psy2 - psy1)

        xx1 = jnp.maximum(psx1, sx1)
        yy1 = jnp.maximum(psy1, sy1)
        xx2 = jnp.minimum(psx2, sx2)
        yy2 = jnp.minimum(psy2, sy2)
        inter = jnp.maximum(0.0, xx2 - xx1) * jnp.maximum(0.0, yy2 - yy1)
        iou = inter / (parea + areas - inter)
        new_cur = jnp.where(iou <= _NMS_THRESH, cur, neg_inf)
        new_cur = jnp.where(pick, neg_inf, new_cur)

        row = (jnp.where(li == 0, jnp.where(ok, pbx1, 0.0), 0.0)
               + jnp.where(li == 1, jnp.where(ok, pby1, 0.0), 0.0)
               + jnp.where(li == 2, jnp.where(ok, pbx2, 0.0), 0.0)
               + jnp.where(li == 3, jnp.where(ok, pby2, 0.0), 0.0)
               + jnp.where(li == 4, jnp.where(ok, psc, 0.0), 0.0))
        out_ref[0, pl.ds(t, 1), :] = row
        return new_cur

    lax.fori_loop(0, _POST_NMS_TOP_N, nms_step, cur0)


def _layout(obj_levels, del_levels, anchors):
    logit_parts, d_parts = [], ([], [], [], [])
    anc_parts = []
    for (h, w), o, d, off, n in zip(_LEVEL_HW, obj_levels, del_levels, _OFFS, _NPL):
        logit_parts.append(o.reshape(_N_IMGS, _A * h * w))
        d5 = d.reshape(_N_IMGS, _A, 4, h, w)
        for c in range(4):
            d_parts[c].append(d5[:, :, c].reshape(_N_IMGS, _A * h * w))
        a = anchors[:, off:off + n, :].reshape(_N_IMGS, h, w, _A, 4)
        anc_parts.append(jnp.transpose(a, (0, 3, 1, 2, 4)).reshape(_N_IMGS, n, 4))

    pad = _PADDED - _TOTAL

    def cat(parts, pad_val):
        x = jnp.concatenate(parts, axis=1)
        x = jnp.pad(x, ((0, 0), (0, pad)), constant_values=pad_val)
        return x.reshape(_N_IMGS, _ROWS, _LANES)

    logits = cat(logit_parts, -1e30)
    dx, dy, dw, dh = (cat(p, 0.0) for p in d_parts)
    anc = jnp.concatenate(anc_parts, axis=1)
    anc = jnp.pad(anc, ((0, 0), (0, pad), (0, 0)))
    ax1, ay1, ax2, ay2 = (anc[..., c].reshape(_N_IMGS, _ROWS, _LANES) for c in range(4))
    return logits, dx, dy, dw, dh, ax1, ay1, ax2, ay2


def kernel(obj_l0, obj_l1, obj_l2, obj_l3, obj_l4,
           del_l0, del_l1, del_l2, del_l3, del_l4,
           anchors, image_sizes):
    obj_levels = [obj_l0, obj_l1, obj_l2, obj_l3, obj_l4]
    del_levels = [del_l0, del_l1, del_l2, del_l3, del_l4]
    planes = _layout(obj_levels, del_levels, anchors)
    logits3d = planes[0]

    img_spec = pl.BlockSpec((1, _ROWS, _LANES), lambda i: (i, 0, 0))

    plane3 = jax.ShapeDtypeStruct((_N_IMGS, _ROWS, _LANES), jnp.float32)
    sel, bx1p, by1p, bx2p, by2p, mxp = pl.pallas_call(
        _prep_body,
        grid=(_N_IMGS,),
        in_specs=[pl.BlockSpec(memory_space=pltpu.SMEM)] + [img_spec] * 9,
        out_specs=[img_spec] * 5 + [pl.BlockSpec((1, 8, _LANES), lambda i: (i, 0, 0))],
        out_shape=[jax.ShapeDtypeStruct((_N_IMGS, _ROWS, _LANES), jnp.int32),
                   plane3, plane3, plane3, plane3,
                   jax.ShapeDtypeStruct((_N_IMGS, 8, _LANES), jnp.float32)],
    )(image_sizes, *planes)

    flat = lambda x: x.reshape(_N_IMGS * _PADDED)
    rank = jnp.asarray(_RANK_CONST)

    cw_f32 = jax.ShapeDtypeStruct((_N_IMGS * _CW,), jnp.float32)
    cw_i32 = jax.ShapeDtypeStruct((_N_IMGS * _CW,), jnp.int32)
    sc_call = pl.kernel(
        _sc_compact_body,
        mesh=plsc.VectorSubcoreMesh(core_axis_name="c", subcore_axis_name="s"),
        compiler_params=pltpu.CompilerParams(needs_layout_passes=False),
        out_type=[cw_i32, cw_f32, cw_f32, cw_f32, cw_f32, cw_f32, cw_i32],
        scratch_types=[
            pltpu.VMEM((_CHUNK,), jnp.int32),
            pltpu.VMEM((_IDXBUF,), jnp.int32),
            pltpu.VMEM((_SLOTS_PER_SUB,), jnp.int32),
            pltpu.VMEM((_SLOTS_PER_SUB,), jnp.int32),
            pltpu.VMEM((_SLOTS_PER_SUB,), jnp.int32),
            pltpu.VMEM((5 * _SLOTS_PER_SUB,), jnp.float32),
            pltpu.VMEM((_SLOTS_PER_SUB,), jnp.int32),
            pltpu.SMEM((1,), jnp.int32),
            pltpu.SemaphoreType.DMA,
        ],
    )
    gidx, gbx1, gby1, gbx2, gby2, glog, grank = sc_call(
        flat(sel), flat(bx1p), flat(by1p), flat(bx2p), flat(by2p),
        flat(logits3d), rank)

    c3 = lambda x: x.reshape(_N_IMGS, _CROWS, _LANES)
    cimg = pl.BlockSpec((1, _CROWS, _LANES), lambda i: (i, 0, 0))
    out = pl.pallas_call(
        _nms_body,
        grid=(_N_IMGS,),
        in_specs=[pl.BlockSpec((1, 8, _LANES), lambda i: (i, 0, 0))] + [cimg] * 7,
        out_specs=pl.BlockSpec((1, _POST_NMS_TOP_N, _LANES), lambda i: (i, 0, 0)),
        out_shape=jax.ShapeDtypeStruct((_N_IMGS, _POST_NMS_TOP_N, _LANES), jnp.float32),
    )(mxp, c3(gidx), c3(gbx1), c3(gby1), c3(gbx2), c3(gby2), c3(glog), c3(grank))

    return out[:, :, 0:4], out[:, :, 4]
